# Initial kernel scaffold; baseline (speedup 1.0000x reference)
import jax
import jax.numpy as jnp
from jax.experimental import pallas as pl


def kernel(x_E, x_H, x_S, W, edge_index):
    raise NotImplementedError("write your pallas kernel here")


# speedup vs baseline: 23.4651x; 23.4651x over previous
import functools

import jax
import jax.numpy as jnp
from jax import lax
from jax.experimental import pallas as pl
from jax.experimental.pallas import tpu as pltpu
from jax.experimental.pallas import tpu_sc as plsc

_N = 50000
_D = 128
_C = 40
_E = 800000

_NC = 2
_NS = 16
_NW = _NC * _NS

_CHUNK = 128
_NCHUNK = 196
_EPT = _CHUNK * _NCHUNK
_E_PAD = _NW * _EPT
_N_PAD = 50048
_ZROWS = _N_PAD // _NS
_ZFULL = _ZROWS // _CHUNK
_ZTAIL = _ZROWS - _ZFULL * _CHUNK

_BLK = 1000


def _h_body(xe_ref, xh_ref, xs_ref, w_ref, h_ref):
    xe = xe_ref[...]
    xh = xh_ref[...]
    xs = xs_ref[...]

    nh = jnp.sqrt(jnp.sum(xh * xh, axis=1, keepdims=True))
    nhc = jnp.clip(nh, 1e-15, 1.0 - 1e-5)
    artanh = 0.5 * jnp.log((1.0 + nhc) / (1.0 - nhc))
    xh_l = artanh * xh / jnp.maximum(nh, 1e-15)

    ns = jnp.sqrt(jnp.sum(xs * xs, axis=1, keepdims=True))
    v1 = ns / (1.0 + jnp.sqrt(1.0 + ns * ns))
    v2 = v1 / (1.0 + jnp.sqrt(1.0 + v1 * v1))
    t2 = v2 * v2
    poly = 1.0 + t2 * (-1.0 / 3 + t2 * (1.0 / 5 + t2 * (-1.0 / 7 + t2 * (
        1.0 / 9 + t2 * (-1.0 / 11 + t2 * (1.0 / 13))))))
    atan_ns = 4.0 * v2 * poly
    xs_l = atan_ns * xs / jnp.maximum(ns, 1e-15)

    h = jnp.dot(xe, w_ref[0:_D, :], preferred_element_type=jnp.float32)
    h += jnp.dot(xh_l, w_ref[_D:2 * _D, :], preferred_element_type=jnp.float32)
    h += jnp.dot(xs_l, w_ref[2 * _D:3 * _D, :], preferred_element_type=jnp.float32)
    h_ref[...] = h


def _compute_h(x_E, x_H, x_S, W):
    grid = (_N // _BLK,)
    xspec = pl.BlockSpec((_BLK, _D), lambda i: (i, 0))
    return pl.pallas_call(
        _h_body,
        grid=grid,
        in_specs=[xspec, xspec, xspec, pl.BlockSpec((3 * _D, _C), lambda i: (0, 0))],
        out_specs=pl.BlockSpec((_BLK, _C), lambda i: (i, 0)),
        out_shape=jax.ShapeDtypeStruct((_N, _C), jnp.float32),
    )(x_E, x_H, x_S, W)


def _deg_body(col_hbm, deg_hbm, idx_v, ones_v, zero_v, deg_sh):
    cid = lax.axis_index("c")
    sid = lax.axis_index("s")
    base = (cid * _NS + sid) * _EPT

    z16 = jnp.zeros((16,), jnp.float32)
    o16 = jnp.ones((16,), jnp.float32)
    for i in range(_CHUNK // 16):
        ones_v[pl.ds(i * 16, 16)] = o16
        zero_v[pl.ds(i * 16, 16)] = z16

    def zloop(t, carry):
        pltpu.sync_copy(zero_v, deg_sh.at[pl.ds(sid * _ZROWS + t * _CHUNK, _CHUNK)])
        return carry
    lax.fori_loop(0, _ZFULL, zloop, 0)
    pltpu.sync_copy(zero_v.at[pl.ds(0, _ZTAIL)],
                    deg_sh.at[pl.ds(sid * _ZROWS + _ZFULL * _CHUNK, _ZTAIL)])
    plsc.subcore_barrier()

    def chunk(j, carry):
        pltpu.sync_copy(col_hbm.at[pl.ds(base + j * _CHUNK, _CHUNK)], idx_v)
        pltpu.sync_copy(ones_v, deg_sh.at[idx_v], add=True)
        return carry
    lax.fori_loop(0, _NCHUNK, chunk, 0)
    plsc.subcore_barrier()

    pltpu.sync_copy(deg_sh.at[pl.ds(sid * _ZROWS, _ZROWS)],
                    deg_hbm.at[cid, pl.ds(sid * _ZROWS, _ZROWS)])


def _compute_deg(col_pad):
    mesh = plsc.VectorSubcoreMesh(core_axis_name="c", subcore_axis_name="s")
    f = pl.kernel(
        _deg_body,
        out_type=jax.ShapeDtypeStruct((_NC, _N_PAD), jnp.float32),
        mesh=mesh,
        scratch_types=[
            pltpu.VMEM((_CHUNK,), jnp.int32),
            pltpu.VMEM((_CHUNK,), jnp.float32),
            pltpu.VMEM((_CHUNK,), jnp.float32),
            pltpu.VMEM_SHARED((_N_PAD,), jnp.float32),
        ],
        compiler_params=pltpu.CompilerParams(use_tc_tiling_on_sc=False),
    )
    return f(col_pad)


def _g_body(h_ref, deg_ref, g_ref):
    deg = deg_ref[:, 0] + deg_ref[:, 1] + 1.0
    dinv = lax.rsqrt(deg)
    g_ref[...] = h_ref[...] * dinv[:, None]


def _compute_g(h, degp_t):
    grid = (_N // _BLK,)
    return pl.pallas_call(
        _g_body,
        grid=grid,
        in_specs=[pl.BlockSpec((_BLK, _C), lambda i: (i, 0)),
                  pl.BlockSpec((_BLK, _NC), lambda i: (i, 0))],
        out_specs=pl.BlockSpec((_BLK, _C), lambda i: (i, 0)),
        out_shape=jax.ShapeDtypeStruct((_N, _C), jnp.float32),
    )(h, degp_t)


def _agg_body(row_hbm, col_hbm, g_hbm, s_hbm, ridx_v, cidx_v, rows_v, sem, acc_sh):
    cid = lax.axis_index("c")
    sid = lax.axis_index("s")
    base = (cid * _NS + sid) * _EPT

    z16 = jnp.zeros((16,), jnp.float32)

    def zrow(i, carry):
        rows_v[i, pl.ds(0, 16)] = z16
        rows_v[i, pl.ds(16, 16)] = z16
        rows_v[i, pl.ds(_C - 16, 16)] = z16
        return carry
    lax.fori_loop(0, _CHUNK, zrow, 0)

    def zacc(t, carry):
        pltpu.sync_copy(rows_v, acc_sh.at[pl.ds(sid * _ZROWS + t * _CHUNK, _CHUNK)])
        return carry
    lax.fori_loop(0, _ZFULL, zacc, 0)
    pltpu.sync_copy(rows_v.at[pl.ds(0, _ZTAIL)],
                    acc_sh.at[pl.ds(sid * _ZROWS + _ZFULL * _CHUNK, _ZTAIL)])
    plsc.subcore_barrier()

    def chunk(j, carry):
        off = base + j * _CHUNK
        pltpu.sync_copy(row_hbm.at[pl.ds(off, _CHUNK)], ridx_v)
        pltpu.sync_copy(col_hbm.at[pl.ds(off, _CHUNK)], cidx_v)
        pltpu.async_copy(g_hbm.at[ridx_v], rows_v, sem).wait()
        pltpu.sync_copy(rows_v, acc_sh.at[cidx_v], add=True)
        return carry
    lax.fori_loop(0, _NCHUNK, chunk, 0)
    plsc.subcore_barrier()

    pltpu.sync_copy(acc_sh.at[pl.ds(sid * _ZROWS, _ZROWS)],
                    s_hbm.at[cid, pl.ds(sid * _ZROWS, _ZROWS)])


def _compute_s(row_pad, col_pad, g):
    mesh = plsc.VectorSubcoreMesh(core_axis_name="c", subcore_axis_name="s")
    f = pl.kernel(
        _agg_body,
        out_type=jax.ShapeDtypeStruct((_NC, _N_PAD, _C), jnp.float32),
        mesh=mesh,
        scratch_types=[
            pltpu.VMEM((_CHUNK,), jnp.int32),
            pltpu.VMEM((_CHUNK,), jnp.int32),
            pltpu.VMEM((_CHUNK, _C), jnp.float32),
            pltpu.SemaphoreType.DMA,
            pltpu.VMEM_SHARED((_N_PAD, _C), jnp.float32),
        ],
        compiler_params=pltpu.CompilerParams(use_tc_tiling_on_sc=False),
    )
    return f(row_pad, col_pad, g)


def _out_body(s_ref, g_ref, deg_ref, o_ref):
    deg = deg_ref[:, 0] + deg_ref[:, 1] + 1.0
    dinv = lax.rsqrt(deg)
    o_ref[...] = (s_ref[0] + s_ref[1] + g_ref[...]) * dinv[:, None]


def _compute_out(s, g, degp_t):
    grid = (_N // _BLK,)
    return pl.pallas_call(
        _out_body,
        grid=grid,
        in_specs=[pl.BlockSpec((_NC, _BLK, _C), lambda i: (0, i, 0)),
                  pl.BlockSpec((_BLK, _C), lambda i: (i, 0)),
                  pl.BlockSpec((_BLK, _NC), lambda i: (i, 0))],
        out_specs=pl.BlockSpec((_BLK, _C), lambda i: (i, 0)),
        out_shape=jax.ShapeDtypeStruct((_N, _C), jnp.float32),
    )(s, g, degp_t)


def kernel(x_E, x_H, x_S, W, edge_index):
    npad = _E_PAD - _E
    row_pad = jnp.concatenate([edge_index[0], jnp.zeros((npad,), jnp.int32)])
    col_pad = jnp.concatenate([edge_index[1], jnp.full((npad,), _N, jnp.int32)])

    h = _compute_h(x_E, x_H, x_S, W)
    degp = _compute_deg(col_pad)
    degp_t = degp.T
    g = _compute_g(h, degp_t)
    s = _compute_s(row_pad, col_pad, g)
    return _compute_out(s, g, degp_t)


# speedup vs baseline: 33.3056x; 1.4194x over previous
import jax
import jax.numpy as jnp
from jax import lax
from jax.experimental import pallas as pl
from jax.experimental.pallas import tpu as pltpu
from jax.experimental.pallas import tpu_sc as plsc

_N = 50000
_D = 128
_C = 40
_E = 800000

_NC = 2
_NS = 16
_NW = _NC * _NS

_CHR = _C // _NC
_CH = 24

_CHUNK = 128
_NCHUNK = 196
_EPT = _CHUNK * _NCHUNK
_E_PAD = _NW * _EPT
_TOTCHUNK = _E_PAD // _CHUNK
_ACHUNK = _TOTCHUNK // _NS
_APHASE = 7
_APC = _ACHUNK // _APHASE

_N_PAD = 50048
_ZROWS = _N_PAD // _NS
_ZFULL = _ZROWS // _CHUNK
_ZTAIL = _ZROWS - _ZFULL * _CHUNK

_NBUF = 4
_BLK = 1000


def _h_body(xe_ref, xh_ref, xs_ref, w_ref, h_ref):
    xe = xe_ref[...]
    xh = xh_ref[...]
    xs = xs_ref[...]

    nh = jnp.sqrt(jnp.sum(xh * xh, axis=1, keepdims=True))
    nhc = jnp.clip(nh, 1e-15, 1.0 - 1e-5)
    artanh = 0.5 * jnp.log((1.0 + nhc) / (1.0 - nhc))
    xh_l = artanh * xh / jnp.maximum(nh, 1e-15)

    ns = jnp.sqrt(jnp.sum(xs * xs, axis=1, keepdims=True))
    v1 = ns / (1.0 + jnp.sqrt(1.0 + ns * ns))
    v2 = v1 / (1.0 + jnp.sqrt(1.0 + v1 * v1))
    t2 = v2 * v2
    poly = 1.0 + t2 * (-1.0 / 3 + t2 * (1.0 / 5 + t2 * (-1.0 / 7 + t2 * (
        1.0 / 9 + t2 * (-1.0 / 11 + t2 * (1.0 / 13))))))
    atan_ns = 4.0 * v2 * poly
    xs_l = atan_ns * xs / jnp.maximum(ns, 1e-15)

    h = jnp.dot(xe, w_ref[0:_D, :], preferred_element_type=jnp.float32)
    h += jnp.dot(xh_l, w_ref[_D:2 * _D, :], preferred_element_type=jnp.float32)
    h += jnp.dot(xs_l, w_ref[2 * _D:3 * _D, :], preferred_element_type=jnp.float32)
    h_ref[...] = h


def _compute_h(x_E, x_H, x_S, W):
    grid = (_N // _BLK,)
    xspec = pl.BlockSpec((_BLK, _D), lambda i: (i, 0))
    return pl.pallas_call(
        _h_body,
        grid=grid,
        in_specs=[xspec, xspec, xspec, pl.BlockSpec((3 * _D, _C), lambda i: (0, 0))],
        out_specs=pl.BlockSpec((_BLK, _C), lambda i: (i, 0)),
        out_shape=jax.ShapeDtypeStruct((_N, _C), jnp.float32),
    )(x_E, x_H, x_S, W)


def _deg_body(col_hbm, deg_hbm, cidx2, ones_v, zero_v, s0, s1, s2, s3, deg_sh):
    cid = lax.axis_index("c")
    sid = lax.axis_index("s")
    wid = cid * _NS + sid
    ssem = [s0, s1, s2, s3]

    z16 = jnp.zeros((16,), jnp.float32)
    o16 = jnp.ones((16,), jnp.float32)
    for i in range(_CHUNK // 16):
        ones_v[pl.ds(i * 16, 16)] = o16
        zero_v[pl.ds(i * 16, 16)] = z16

    pltpu.sync_copy(col_hbm.at[pl.ds(wid * _NCHUNK, _NCHUNK)], cidx2)

    def zloop(t, carry):
        pltpu.sync_copy(zero_v, deg_sh.at[pl.ds(sid * _ZROWS + t * _CHUNK, _CHUNK)])
        return carry
    lax.fori_loop(0, _ZFULL, zloop, 0)
    pltpu.sync_copy(zero_v.at[pl.ds(0, _ZTAIL)],
                    deg_sh.at[pl.ds(sid * _ZROWS + _ZFULL * _CHUNK, _ZTAIL)])
    plsc.subcore_barrier()

    ngrp = _NCHUNK // _NBUF
    def grp(gi, carry):
        descs = []
        for b in range(_NBUF):
            j = gi * _NBUF + b
            descs.append(pltpu.async_copy(
                ones_v, deg_sh.at[cidx2.at[j]], ssem[b], add=True))
        for d in descs:
            d.wait()
        return carry
    lax.fori_loop(0, ngrp, grp, 0)
    plsc.subcore_barrier()

    pltpu.sync_copy(deg_sh.at[pl.ds(sid * _ZROWS, _ZROWS)],
                    deg_hbm.at[cid, pl.ds(sid * _ZROWS, _ZROWS)])


def _compute_deg(col2):
    mesh = plsc.VectorSubcoreMesh(core_axis_name="c", subcore_axis_name="s")
    f = pl.kernel(
        _deg_body,
        out_type=jax.ShapeDtypeStruct((_NC, _N_PAD), jnp.float32),
        mesh=mesh,
        scratch_types=[
            pltpu.VMEM((_NCHUNK, _CHUNK), jnp.int32),
            pltpu.VMEM((_CHUNK,), jnp.float32),
            pltpu.VMEM((_CHUNK,), jnp.float32),
            pltpu.SemaphoreType.DMA,
            pltpu.SemaphoreType.DMA,
            pltpu.SemaphoreType.DMA,
            pltpu.SemaphoreType.DMA,
            pltpu.VMEM_SHARED((_N_PAD,), jnp.float32),
        ],
        compiler_params=pltpu.CompilerParams(use_tc_tiling_on_sc=False),
    )
    return f(col2)


def _g_body(h_ref, deg_ref, g_ref):
    deg = deg_ref[:, 0] + deg_ref[:, 1] + 1.0
    dinv = lax.rsqrt(deg)
    g = h_ref[...] * dinv[:, None]
    pad = jnp.zeros((g.shape[0], _CH - _CHR), jnp.float32)
    g_ref[0] = jnp.concatenate([g[:, 0:_CHR], pad], axis=1)
    g_ref[1] = jnp.concatenate([g[:, _CHR:_C], pad], axis=1)


def _compute_g(h, degp_t):
    grid = (_N // _BLK,)
    return pl.pallas_call(
        _g_body,
        grid=grid,
        in_specs=[pl.BlockSpec((_BLK, _C), lambda i: (i, 0)),
                  pl.BlockSpec((_BLK, _NC), lambda i: (i, 0))],
        out_specs=pl.BlockSpec((_NC, _BLK, _CH), lambda i: (0, i, 0)),
        out_shape=jax.ShapeDtypeStruct((_NC, _N, _CH), jnp.float32),
    )(h, degp_t)


def _agg_body(row_hbm, col_hbm, g2_hbm, s_hbm, ridx2, cidx2,
              r0, r1, r2, r3, g0, g1, g2, g3, acc_sh):
    cid = lax.axis_index("c")
    sid = lax.axis_index("s")
    rows = [r0, r1, r2, r3]
    gsem = [g0, g1, g2, g3]
    gsrc = g2_hbm.at[cid]

    z16 = jnp.zeros((16,), jnp.float32)

    def zrow(i, carry):
        rows[0][i, pl.ds(0, 16)] = z16
        rows[0][i, pl.ds(8, 16)] = z16
        return carry
    lax.fori_loop(0, _CHUNK, zrow, 0)

    def zacc(t, carry):
        pltpu.sync_copy(rows[0], acc_sh.at[pl.ds(sid * _ZROWS + t * _CHUNK, _CHUNK)])
        return carry
    lax.fori_loop(0, _ZFULL, zacc, 0)
    pltpu.sync_copy(rows[0].at[pl.ds(0, _ZTAIL)],
                    acc_sh.at[pl.ds(sid * _ZROWS + _ZFULL * _CHUNK, _ZTAIL)])
    plsc.subcore_barrier()

    for p in range(_APHASE):
        cbase = sid * _ACHUNK + p * _APC
        pltpu.sync_copy(row_hbm.at[pl.ds(cbase, _APC)], ridx2)
        pltpu.sync_copy(col_hbm.at[pl.ds(cbase, _APC)], cidx2)

        ngrp = _APC // _NBUF
        def grp(gi, carry):
            descs = []
            for b in range(_NBUF):
                j = gi * _NBUF + b
                descs.append(pltpu.async_copy(
                    gsrc.at[ridx2.at[j]], rows[b], gsem[b]))
            for b in range(_NBUF):
                j = gi * _NBUF + b
                descs[b].wait()
                pltpu.sync_copy(rows[b], acc_sh.at[cidx2.at[j]], add=True)
            return carry
        lax.fori_loop(0, ngrp, grp, 0)

    plsc.subcore_barrier()
    pltpu.sync_copy(acc_sh.at[pl.ds(sid * _ZROWS, _ZROWS)],
                    s_hbm.at[cid, pl.ds(sid * _ZROWS, _ZROWS)])


def _compute_s(row2, col2, g2):
    mesh = plsc.VectorSubcoreMesh(core_axis_name="c", subcore_axis_name="s")
    f = pl.kernel(
        _agg_body,
        out_type=jax.ShapeDtypeStruct((_NC, _N_PAD, _CH), jnp.float32),
        mesh=mesh,
        scratch_types=[
            pltpu.VMEM((_APC, _CHUNK), jnp.int32),
            pltpu.VMEM((_APC, _CHUNK), jnp.int32),
            pltpu.VMEM((_CHUNK, _CH), jnp.float32),
            pltpu.VMEM((_CHUNK, _CH), jnp.float32),
            pltpu.VMEM((_CHUNK, _CH), jnp.float32),
            pltpu.VMEM((_CHUNK, _CH), jnp.float32),
            pltpu.SemaphoreType.DMA,
            pltpu.SemaphoreType.DMA,
            pltpu.SemaphoreType.DMA,
            pltpu.SemaphoreType.DMA,
            pltpu.VMEM_SHARED((_N_PAD, _CH), jnp.float32),
        ],
        compiler_params=pltpu.CompilerParams(use_tc_tiling_on_sc=False),
    )
    return f(row2, col2, g2)


def _out_body(s_ref, g_ref, deg_ref, o_ref):
    deg = deg_ref[:, 0] + deg_ref[:, 1] + 1.0
    dinv = lax.rsqrt(deg)
    tot = jnp.concatenate([(s_ref[0] + g_ref[0])[:, 0:_CHR],
                           (s_ref[1] + g_ref[1])[:, 0:_CHR]], axis=1)
    o_ref[...] = tot * dinv[:, None]


def _compute_out(s, g2, degp_t):
    grid = (_N // _BLK,)
    return pl.pallas_call(
        _out_body,
        grid=grid,
        in_specs=[pl.BlockSpec((_NC, _BLK, _CH), lambda i: (0, i, 0)),
                  pl.BlockSpec((_NC, _BLK, _CH), lambda i: (0, i, 0)),
                  pl.BlockSpec((_BLK, _NC), lambda i: (i, 0))],
        out_specs=pl.BlockSpec((_BLK, _C), lambda i: (i, 0)),
        out_shape=jax.ShapeDtypeStruct((_N, _C), jnp.float32),
    )(s, g2, degp_t)


def kernel(x_E, x_H, x_S, W, edge_index):
    npad = _E_PAD - _E
    row2 = jnp.concatenate(
        [edge_index[0], jnp.zeros((npad,), jnp.int32)]).reshape(
            _TOTCHUNK, _CHUNK)
    col2 = jnp.concatenate(
        [edge_index[1], jnp.full((npad,), _N, jnp.int32)]).reshape(
            _TOTCHUNK, _CHUNK)

    h = _compute_h(x_E, x_H, x_S, W)
    degp = _compute_deg(col2)
    degp_t = degp.T
    g2 = _compute_g(h, degp_t)
    s = _compute_s(row2, col2, g2)
    return _compute_out(s, g2, degp_t)


# speedup vs baseline: 35.3091x; 1.0602x over previous
import jax
import jax.numpy as jnp
from jax import lax
from jax.experimental import pallas as pl
from jax.experimental.pallas import tpu as pltpu
from jax.experimental.pallas import tpu_sc as plsc

_N = 50000
_D = 128
_C = 40
_E = 800000

_NC = 2
_NS = 16
_NW = _NC * _NS

_CHR = _C // _NC
_CH = 24

_CHUNK = 128
_NCHUNK = 196
_EPT = _CHUNK * _NCHUNK
_E_PAD = _NW * _EPT
_TOTCHUNK = _E_PAD // _CHUNK
_ACHUNK = _TOTCHUNK // _NS
_APHASE = 7
_APC = _ACHUNK // _APHASE

_N_PAD = 50048
_ZROWS = _N_PAD // _NS
_ZFULL = _ZROWS // _CHUNK
_ZTAIL = _ZROWS - _ZFULL * _CHUNK

_NBUF = 4
_BLK = 1000


def _h_body(xe_ref, xh_ref, xs_ref, w_ref, h_ref):
    xe = xe_ref[...]
    xh = xh_ref[...]
    xs = xs_ref[...]

    nh = jnp.sqrt(jnp.sum(xh * xh, axis=1, keepdims=True))
    nhc = jnp.clip(nh, 1e-15, 1.0 - 1e-5)
    artanh = 0.5 * jnp.log((1.0 + nhc) / (1.0 - nhc))
    xh_l = artanh * xh / jnp.maximum(nh, 1e-15)

    ns = jnp.sqrt(jnp.sum(xs * xs, axis=1, keepdims=True))
    v1 = ns / (1.0 + jnp.sqrt(1.0 + ns * ns))
    v2 = v1 / (1.0 + jnp.sqrt(1.0 + v1 * v1))
    t2 = v2 * v2
    poly = 1.0 + t2 * (-1.0 / 3 + t2 * (1.0 / 5 + t2 * (-1.0 / 7 + t2 * (
        1.0 / 9 + t2 * (-1.0 / 11 + t2 * (1.0 / 13))))))
    atan_ns = 4.0 * v2 * poly
    xs_l = atan_ns * xs / jnp.maximum(ns, 1e-15)

    h = jnp.dot(xe, w_ref[0:_D, :], preferred_element_type=jnp.float32)
    h += jnp.dot(xh_l, w_ref[_D:2 * _D, :], preferred_element_type=jnp.float32)
    h += jnp.dot(xs_l, w_ref[2 * _D:3 * _D, :], preferred_element_type=jnp.float32)
    h_ref[...] = h


def _compute_h(x_E, x_H, x_S, W):
    grid = (_N // _BLK,)
    xspec = pl.BlockSpec((_BLK, _D), lambda i: (i, 0))
    return pl.pallas_call(
        _h_body,
        grid=grid,
        in_specs=[xspec, xspec, xspec, pl.BlockSpec((3 * _D, _C), lambda i: (0, 0))],
        out_specs=pl.BlockSpec((_BLK, _C), lambda i: (i, 0)),
        out_shape=jax.ShapeDtypeStruct((_N, _C), jnp.float32),
    )(x_E, x_H, x_S, W)


def _deg_body(col_hbm, deg_hbm, cidx2, ones_v, zero_v, s0, s1, s2, s3, deg_sh):
    cid = lax.axis_index("c")
    sid = lax.axis_index("s")
    wid = cid * _NS + sid
    ssem = [s0, s1, s2, s3]

    z16 = jnp.zeros((16,), jnp.float32)
    o16 = jnp.ones((16,), jnp.float32)
    for i in range(_CHUNK // 16):
        ones_v[pl.ds(i * 16, 16)] = o16
        zero_v[pl.ds(i * 16, 16)] = z16

    pltpu.sync_copy(col_hbm.at[pl.ds(wid * _NCHUNK, _NCHUNK)], cidx2)

    def zloop(t, carry):
        pltpu.sync_copy(zero_v, deg_sh.at[pl.ds(sid * _ZROWS + t * _CHUNK, _CHUNK)])
        return carry
    lax.fori_loop(0, _ZFULL, zloop, 0)
    pltpu.sync_copy(zero_v.at[pl.ds(0, _ZTAIL)],
                    deg_sh.at[pl.ds(sid * _ZROWS + _ZFULL * _CHUNK, _ZTAIL)])
    plsc.subcore_barrier()

    ngrp = _NCHUNK // _NBUF
    def grp(gi, carry):
        descs = []
        for b in range(_NBUF):
            j = gi * _NBUF + b
            descs.append(pltpu.async_copy(
                ones_v, deg_sh.at[cidx2.at[j]], ssem[b], add=True))
        for d in descs:
            d.wait()
        return carry
    lax.fori_loop(0, ngrp, grp, 0)
    plsc.subcore_barrier()

    pltpu.sync_copy(deg_sh.at[pl.ds(sid * _ZROWS, _ZROWS)],
                    deg_hbm.at[cid, pl.ds(sid * _ZROWS, _ZROWS)])


def _compute_deg(col2):
    mesh = plsc.VectorSubcoreMesh(core_axis_name="c", subcore_axis_name="s")
    f = pl.kernel(
        _deg_body,
        out_type=jax.ShapeDtypeStruct((_NC, _N_PAD), jnp.float32),
        mesh=mesh,
        scratch_types=[
            pltpu.VMEM((_NCHUNK, _CHUNK), jnp.int32),
            pltpu.VMEM((_CHUNK,), jnp.float32),
            pltpu.VMEM((_CHUNK,), jnp.float32),
            pltpu.SemaphoreType.DMA,
            pltpu.SemaphoreType.DMA,
            pltpu.SemaphoreType.DMA,
            pltpu.SemaphoreType.DMA,
            pltpu.VMEM_SHARED((_N_PAD,), jnp.float32),
        ],
        compiler_params=pltpu.CompilerParams(use_tc_tiling_on_sc=False),
    )
    return f(col2)


def _g_body(h_ref, deg_ref, g_ref):
    deg = deg_ref[:, 0] + deg_ref[:, 1] + 1.0
    dinv = lax.rsqrt(deg)
    g = h_ref[...] * dinv[:, None]
    pad = jnp.zeros((g.shape[0], _CH - _CHR), jnp.float32)
    g_ref[0] = jnp.concatenate([g[:, 0:_CHR], pad], axis=1)
    g_ref[1] = jnp.concatenate([g[:, _CHR:_C], pad], axis=1)


def _compute_g(h, degp_t):
    grid = (_N // _BLK,)
    return pl.pallas_call(
        _g_body,
        grid=grid,
        in_specs=[pl.BlockSpec((_BLK, _C), lambda i: (i, 0)),
                  pl.BlockSpec((_BLK, _NC), lambda i: (i, 0))],
        out_specs=pl.BlockSpec((_NC, _BLK, _CH), lambda i: (0, i, 0)),
        out_shape=jax.ShapeDtypeStruct((_NC, _N, _CH), jnp.float32),
    )(h, degp_t)


_ABUF = 8


def _agg_body(row_hbm, col_hbm, g2_hbm, s_hbm, ridx2, cidx2,
              r0, r1, r2, r3, r4, r5, r6, r7,
              g0, g1, g2, g3, g4, g5, g6, g7, acc_sh):
    cid = lax.axis_index("c")
    sid = lax.axis_index("s")
    rows = [r0, r1, r2, r3, r4, r5, r6, r7]
    gsem = [g0, g1, g2, g3, g4, g5, g6, g7]
    gsrc = g2_hbm.at[cid]

    z16 = jnp.zeros((16,), jnp.float32)

    def zrow(i, carry):
        rows[0][i, pl.ds(0, 16)] = z16
        rows[0][i, pl.ds(8, 16)] = z16
        return carry
    lax.fori_loop(0, _CHUNK, zrow, 0)

    def zacc(t, carry):
        pltpu.sync_copy(rows[0], acc_sh.at[pl.ds(sid * _ZROWS + t * _CHUNK, _CHUNK)])
        return carry
    lax.fori_loop(0, _ZFULL, zacc, 0)
    pltpu.sync_copy(rows[0].at[pl.ds(0, _ZTAIL)],
                    acc_sh.at[pl.ds(sid * _ZROWS + _ZFULL * _CHUNK, _ZTAIL)])
    plsc.subcore_barrier()

    for p in range(_APHASE):
        cbase = sid * _ACHUNK + p * _APC
        pltpu.sync_copy(row_hbm.at[pl.ds(cbase, _APC)], ridx2)
        pltpu.sync_copy(col_hbm.at[pl.ds(cbase, _APC)], cidx2)

        ngrp = _APC // _ABUF
        def grp(gi, carry):
            descs = []
            for b in range(_ABUF):
                j = gi * _ABUF + b
                descs.append(pltpu.async_copy(
                    gsrc.at[ridx2.at[j]], rows[b], gsem[b]))
            for b in range(_ABUF):
                j = gi * _ABUF + b
                descs[b].wait()
                pltpu.sync_copy(rows[b], acc_sh.at[cidx2.at[j]], add=True)
            return carry
        lax.fori_loop(0, ngrp, grp, 0)

    plsc.subcore_barrier()
    pltpu.sync_copy(acc_sh.at[pl.ds(sid * _ZROWS, _ZROWS)],
                    s_hbm.at[cid, pl.ds(sid * _ZROWS, _ZROWS)])


def _compute_s(row2, col2, g2):
    mesh = plsc.VectorSubcoreMesh(core_axis_name="c", subcore_axis_name="s")
    f = pl.kernel(
        _agg_body,
        out_type=jax.ShapeDtypeStruct((_NC, _N_PAD, _CH), jnp.float32),
        mesh=mesh,
        scratch_types=[
            pltpu.VMEM((_APC, _CHUNK), jnp.int32),
            pltpu.VMEM((_APC, _CHUNK), jnp.int32),
            *([pltpu.VMEM((_CHUNK, _CH), jnp.float32)] * _ABUF),
            *([pltpu.SemaphoreType.DMA] * _ABUF),
            pltpu.VMEM_SHARED((_N_PAD, _CH), jnp.float32),
        ],
        compiler_params=pltpu.CompilerParams(use_tc_tiling_on_sc=False),
    )
    return f(row2, col2, g2)


def _out_body(s_ref, g_ref, deg_ref, o_ref):
    deg = deg_ref[:, 0] + deg_ref[:, 1] + 1.0
    dinv = lax.rsqrt(deg)
    tot = jnp.concatenate([(s_ref[0] + g_ref[0])[:, 0:_CHR],
                           (s_ref[1] + g_ref[1])[:, 0:_CHR]], axis=1)
    o_ref[...] = tot * dinv[:, None]


def _compute_out(s, g2, degp_t):
    grid = (_N // _BLK,)
    return pl.pallas_call(
        _out_body,
        grid=grid,
        in_specs=[pl.BlockSpec((_NC, _BLK, _CH), lambda i: (0, i, 0)),
                  pl.BlockSpec((_NC, _BLK, _CH), lambda i: (0, i, 0)),
                  pl.BlockSpec((_BLK, _NC), lambda i: (i, 0))],
        out_specs=pl.BlockSpec((_BLK, _C), lambda i: (i, 0)),
        out_shape=jax.ShapeDtypeStruct((_N, _C), jnp.float32),
    )(s, g2, degp_t)


def kernel(x_E, x_H, x_S, W, edge_index):
    npad = _E_PAD - _E
    row2 = jnp.concatenate(
        [edge_index[0], jnp.zeros((npad,), jnp.int32)]).reshape(
            _TOTCHUNK, _CHUNK)
    col2 = jnp.concatenate(
        [edge_index[1], jnp.full((npad,), _N, jnp.int32)]).reshape(
            _TOTCHUNK, _CHUNK)

    h = _compute_h(x_E, x_H, x_S, W)
    degp = _compute_deg(col2)
    degp_t = degp.T
    g2 = _compute_g(h, degp_t)
    s = _compute_s(row2, col2, g2)
    return _compute_out(s, g2, degp_t)
